# baseline (speedup 1.0000x reference)
import functools

import jax
import jax.numpy as jnp
from jax import lax
from jax.experimental import pallas as pl
from jax.experimental.pallas import tpu as pltpu
from jax.experimental.pallas import tpu_sc as plsc

_NC = 2
_NS = 16
_LANES = 16
_C = 64
_IB = 32
_ZR = 16
_LW = 128
_TRASH = 16


def _pad_cols(a, width):
    if a.shape[1] == width:
        return a
    return jnp.concatenate(
        [a, jnp.zeros((a.shape[0], width - a.shape[1]), a.dtype)], axis=1)


def _bn_body(*refs, nparts, leaky, dl):
    parts = refs[:nparts]
    g, b = refs[nparts], refs[nparts + 1]
    outs = refs[nparts + 2:]
    h = jnp.concatenate([p[...] for p in parts], axis=1)
    m = jnp.mean(h, axis=0, keepdims=True)
    v = jnp.mean((h - m) ** 2, axis=0, keepdims=True)
    hn = (h - m) * lax.rsqrt(v + 1e-5) * g[...] + b[...]
    if leaky:
        hn = jnp.where(hn >= 0, hn, 0.01 * hn)
    if len(outs) == 1:
        outs[0][...] = _pad_cols(hn, outs[0].shape[1])
    else:
        lw = outs[0].shape[1]
        outs[0][...] = _pad_cols(hn[:, :dl], lw)
        outs[1][...] = _pad_cols(hn[:, dl:], lw)


def _eproj_body(*refs, dl):
    a_ref, w_ref, b_ref = refs[:3]
    outs = refs[3:]
    e = jnp.dot(a_ref[...], w_ref[...], preferred_element_type=jnp.float32)
    e = e + b_ref[...]
    if len(outs) == 1:
        outs[0][...] = _pad_cols(e, outs[0].shape[1])
    else:
        lw = outs[0].shape[1]
        outs[0][...] = _pad_cols(e[:, :dl], lw)
        outs[1][...] = _pad_cols(e[:, dl:], lw)


def _mlp_core(hn, ag, eps, w1, b1, g, b, w2, b2, out):
    z = (1.0 + eps) * hn + ag
    z = jnp.dot(z, w1[...], preferred_element_type=jnp.float32) + b1[...]
    m = jnp.mean(z, axis=0, keepdims=True)
    v = jnp.mean((z - m) ** 2, axis=0, keepdims=True)
    z = (z - m) * lax.rsqrt(v + 1e-5) * g[...] + b[...]
    z = jnp.where(z >= 0, z, 0.01 * z)
    out[...] = jnp.dot(z, w2[...], preferred_element_type=jnp.float32) + b2[...]


def _mlp_body_split(hnl, hnr, agl, agr, w1, b1, g, b, w2, b2, eps_ref, out,
                    *, dl):
    hn = jnp.concatenate([hnl[:, :dl], hnr[:, :dl]], axis=1)
    ag = jnp.concatenate([agl[:, :dl], agr[:, :dl]], axis=1)
    _mlp_core(hn, ag, eps_ref[0, 0], w1, b1, g, b, w2, b2, out)


def _mlp_body_full(hn_ref, aga, agb, w1, b1, g, b, w2, b2, eps_ref, out,
                   *, dl):
    hn = hn_ref[:, :dl]
    ag = aga[:, :dl] + agb[:, :dl]
    _mlp_core(hn, ag, eps_ref[0, 0], w1, b1, g, b, w2, b2, out)


def _final_body(*refs):
    parts, w, b, out = refs[:-3], refs[-3], refs[-2], refs[-1]
    h = jnp.concatenate([p[...] for p in parts], axis=1)
    out[...] = jnp.dot(h, w[...], preferred_element_type=jnp.float32) + b[...]


def _zero_accum(s, zb_v, agg_sh, rows_per_tile, rows_rem, row0):
    zero = jnp.zeros((_LANES,), jnp.float32)

    def zrow(r, _):
        for j in range(_LW // _LANES):
            zb_v[r, pl.ds(j * _LANES, _LANES)] = zero
        return 0

    lax.fori_loop(0, _ZR, zrow, 0)

    def zcp(k, _):
        pltpu.sync_copy(zb_v, agg_sh.at[pl.ds(row0 + k * _ZR, _ZR)])
        return 0

    lax.fori_loop(0, rows_per_tile // _ZR, zcp, 0)

    if rows_rem:
        @pl.when(s == 0)
        def _():
            def zcp_rem(k, _):
                pltpu.sync_copy(
                    zb_v, agg_sh.at[pl.ds(rows_per_tile * _NS + k * _ZR, _ZR)])
                return 0
            lax.fori_loop(0, rows_rem // _ZR, zcp_rem, 0)


def _run_edges(hn_h, e_h, agg_sh, idx_src, idx_dst, src_v, dst_v,
               msgs, esems, gsems, ssems, ebase, nblocks):
    def block(bb, _):
        pltpu.sync_copy(idx_src(bb), src_v)
        pltpu.sync_copy(idx_dst(bb), dst_v)
        base = ebase + bb * _IB * _C

        def eload(j, b):
            pltpu.async_copy(e_h.at[pl.ds(base + j * _C, _C)],
                             msgs[b], esems[b])

        def ewait(j, b):
            pltpu.make_async_copy(e_h.at[pl.ds(base + j * _C, _C)],
                                  msgs[b], esems[b]).wait()

        def gissue(j, b):
            pltpu.async_copy(hn_h.at[src_v.at[j]], msgs[b], gsems[b],
                             add=True)

        def gwait(j, b):
            pltpu.make_async_copy(hn_h.at[src_v.at[j]], msgs[b],
                                  gsems[b]).wait()

        def sissue(j, b):
            pltpu.async_copy(msgs[b], agg_sh.at[dst_v.at[j]], ssems[b],
                             add=True)

        def swait(j, b):
            pltpu.make_async_copy(msgs[b], agg_sh.at[dst_v.at[j]],
                                  ssems[b]).wait()

        def relu(b):
            m = msgs[b]

            @plsc.parallel_loop(0, _C, unroll=4)
            def _(r):
                for q in range(_LW // _LANES):
                    sl = pl.ds(q * _LANES, _LANES)
                    m[r, sl] = jnp.maximum(m[r, sl], 0.0)

        eload(0, 0)
        eload(1, 1)
        eload(2, 2)
        ewait(0, 0)
        gissue(0, 0)

        for j in range(4):
            u = j % 4
            gwait(j, u)
            ewait(j + 1, (u + 1) % 4)
            gissue(j + 1, (u + 1) % 4)
            relu(u)
            sissue(j, u)
            if j >= 1:
                swait(j - 1, (u + 3) % 4)
            if j + 3 < _IB:
                eload(j + 3, (u + 3) % 4)

        def quad(t, _):
            for u in range(4):
                j = 4 * t + u
                gwait(j, u)
                ewait(j + 1, (u + 1) % 4)
                gissue(j + 1, (u + 1) % 4)
                relu(u)
                sissue(j, u)
                swait(j - 1, (u + 3) % 4)
                eload(j + 3, (u + 3) % 4)
            return 0

        lax.fori_loop(1, _IB // 4 - 1, quad, 0)

        for j in range(_IB - 4, _IB):
            u = j % 4
            gwait(j, u)
            if j + 1 < _IB:
                ewait(j + 1, (u + 1) % 4)
                gissue(j + 1, (u + 1) % 4)
            relu(u)
            sissue(j, u)
            if j + 3 < _IB:
                swait(j - 1, (u + 3) % 4)
                eload(j + 3, (u + 3) % 4)

        for j in range(_IB - 4, _IB):
            swait(j, j % 4)
        return 0

    lax.fori_loop(0, nblocks, block, 0)


def _sc_body_fsplit(hn_l, hn_r, e_l, e_r, src_r, dst_r, agg_l, agg_r,
                    src_v, dst_v, msg0, msg1, msg2, msg3, zb_v, agg_sh,
                    es0, es1, es2, es3, gs0, gs1, gs2, gs3,
                    ss0, ss1, ss2, ss3, *, n_nodes, steps):
    c = lax.axis_index("c")
    s = lax.axis_index("s")
    rows_per_tile = (n_nodes // (_NS * 8)) * 8
    rows_rem = n_nodes - rows_per_tile * _NS
    row0 = s * rows_per_tile
    ebase = s * (steps * _C)
    nblocks = steps // _IB
    msgs = (msg0, msg1, msg2, msg3)
    esems = (es0, es1, es2, es3)
    gsems = (gs0, gs1, gs2, gs3)
    ssems = (ss0, ss1, ss2, ss3)

    _zero_accum(s, zb_v, agg_sh, rows_per_tile, rows_rem, row0)
    plsc.subcore_barrier()

    def run(hn_h, e_h, agg_h):
        _run_edges(hn_h, e_h, agg_sh,
                   lambda bb: src_r.at[s, bb], lambda bb: dst_r.at[s, bb],
                   src_v, dst_v, msgs, esems, gsems, ssems, ebase, nblocks)
        plsc.subcore_barrier()
        pltpu.sync_copy(agg_sh.at[pl.ds(row0, rows_per_tile)],
                        agg_h.at[pl.ds(row0, rows_per_tile)])
        if rows_rem:
            @pl.when(s == 0)
            def _():
                base = rows_per_tile * _NS
                pltpu.sync_copy(agg_sh.at[pl.ds(base, rows_rem)],
                                agg_h.at[pl.ds(base, rows_rem)])

    @pl.when(c == 0)
    def _():
        run(hn_l, e_l, agg_l)

    @pl.when(c == 1)
    def _():
        run(hn_r, e_r, agg_r)


def _sc_body_esplit(hn, e, src_r, dst_r, agg_a, agg_b,
                    src_v, dst_v, msg0, msg1, msg2, msg3, zb_v, agg_sh,
                    es0, es1, es2, es3, gs0, gs1, gs2, gs3,
                    ss0, ss1, ss2, ss3, *, n_nodes, steps):
    c = lax.axis_index("c")
    s = lax.axis_index("s")
    rows_per_tile = (n_nodes // (_NS * 8)) * 8
    rows_rem = n_nodes - rows_per_tile * _NS
    row0 = s * rows_per_tile
    ebase = c * (_NS * steps * _C) + s * (steps * _C)
    nblocks = steps // _IB
    msgs = (msg0, msg1, msg2, msg3)
    esems = (es0, es1, es2, es3)
    gsems = (gs0, gs1, gs2, gs3)
    ssems = (ss0, ss1, ss2, ss3)

    _zero_accum(s, zb_v, agg_sh, rows_per_tile, rows_rem, row0)
    plsc.subcore_barrier()

    def run(agg_h):
        _run_edges(hn, e, agg_sh,
                   lambda bb: src_r.at[c, s, bb],
                   lambda bb: dst_r.at[c, s, bb],
                   src_v, dst_v, msgs, esems, gsems, ssems, ebase, nblocks)
        plsc.subcore_barrier()
        pltpu.sync_copy(agg_sh.at[pl.ds(row0, rows_per_tile)],
                        agg_h.at[pl.ds(row0, rows_per_tile)])
        if rows_rem:
            @pl.when(s == 0)
            def _():
                base = rows_per_tile * _NS
                pltpu.sync_copy(agg_sh.at[pl.ds(base, rows_rem)],
                                agg_h.at[pl.ds(base, rows_rem)])

    @pl.when(c == 0)
    def _():
        run(agg_a)

    @pl.when(c == 1)
    def _():
        run(agg_b)


def _sc_scratch(n_nodes):
    f32 = jnp.float32
    return [
        pltpu.VMEM((_IB, _C), jnp.int32),
        pltpu.VMEM((_IB, _C), jnp.int32),
        pltpu.VMEM((_C, _LW), f32),
        pltpu.VMEM((_C, _LW), f32),
        pltpu.VMEM((_C, _LW), f32),
        pltpu.VMEM((_C, _LW), f32),
        pltpu.VMEM((_ZR, _LW), f32),
        pltpu.VMEM_SHARED((n_nodes + _TRASH, _LW), f32),
    ] + [pltpu.SemaphoreType.DMA] * 12


def _sc_edge_agg_fsplit(hn_l, hn_r, e_l, e_r, src_r, dst_r, *, n_nodes,
                        n_edges_pad):
    steps = n_edges_pad // (_NS * _C)
    mesh = plsc.VectorSubcoreMesh(core_axis_name="c", subcore_axis_name="s")
    f32 = jnp.float32
    return pl.kernel(
        functools.partial(_sc_body_fsplit, n_nodes=n_nodes, steps=steps),
        out_type=(jax.ShapeDtypeStruct((n_nodes, _LW), f32),
                  jax.ShapeDtypeStruct((n_nodes, _LW), f32)),
        mesh=mesh,
        scratch_types=_sc_scratch(n_nodes),
    )(hn_l, hn_r, e_l, e_r, src_r, dst_r)


def _sc_edge_agg_esplit(hn, e, src_r, dst_r, *, n_nodes, n_edges_pad):
    steps = n_edges_pad // (_NC * _NS * _C)
    mesh = plsc.VectorSubcoreMesh(core_axis_name="c", subcore_axis_name="s")
    f32 = jnp.float32
    return pl.kernel(
        functools.partial(_sc_body_esplit, n_nodes=n_nodes, steps=steps),
        out_type=(jax.ShapeDtypeStruct((n_nodes, _LW), f32),
                  jax.ShapeDtypeStruct((n_nodes, _LW), f32)),
        mesh=mesh,
        scratch_types=_sc_scratch(n_nodes),
    )(hn, e, src_r, dst_r)


def _tc_call(body, out_shapes, *args):
    return pl.pallas_call(body, out_shape=out_shapes)(*args)


def kernel(x, edge_index, edge_attr, params):
    n, d_feat = x.shape
    e_cnt = edge_index.shape[1]
    f32 = jnp.float32

    chunk = _NC * _NS * _IB * _C
    e_pad = ((e_cnt + chunk - 1) // chunk) * chunk
    steps_f = e_pad // (_NS * _C)
    steps_e = e_pad // (_NC * _NS * _C)
    src_flat = jnp.concatenate(
        [edge_index[0].astype(jnp.int32),
         jnp.zeros((e_pad - e_cnt,), jnp.int32)])
    dst_flat = jnp.concatenate(
        [edge_index[1].astype(jnp.int32),
         jnp.full((e_pad - e_cnt,), n, jnp.int32)])
    src_rf = src_flat.reshape(_NS, steps_f // _IB, _IB, _C)
    dst_rf = dst_flat.reshape(_NS, steps_f // _IB, _IB, _C)
    src_re = src_flat.reshape(_NC, _NS, steps_e // _IB, _IB, _C)
    dst_re = dst_flat.reshape(_NC, _NS, steps_e // _IB, _IB, _C)
    ea_pad = jnp.concatenate(
        [edge_attr, jnp.zeros((e_pad - e_cnt, edge_attr.shape[1]), f32)])

    parts = [x]
    for i, p in enumerate(params['layers']):
        din = sum(q.shape[1] for q in parts)
        dl = din // 2
        esplit = (din == _LW)
        n_hn = 1 if esplit else 2
        g2 = p['bn_g'].reshape(1, din)
        b2 = p['bn_b'].reshape(1, din)
        hn_shapes = tuple(jax.ShapeDtypeStruct((n, _LW), f32)
                          for _ in range(n_hn))
        hn_outs = _tc_call(
            functools.partial(_bn_body, nparts=len(parts), leaky=(i > 0),
                              dl=dl),
            hn_shapes, *parts, g2, b2)

        be = 4096
        e_shapes = tuple(jax.ShapeDtypeStruct((e_pad, _LW), f32)
                         for _ in range(n_hn))
        e_outs = pl.pallas_call(
            functools.partial(_eproj_body, dl=dl),
            grid=(e_pad // be,),
            in_specs=[
                pl.BlockSpec((be, edge_attr.shape[1]), lambda j: (j, 0)),
                pl.BlockSpec((edge_attr.shape[1], din), lambda j: (0, 0)),
                pl.BlockSpec((1, din), lambda j: (0, 0)),
            ],
            out_specs=[pl.BlockSpec((be, _LW), lambda j: (j, 0))] * n_hn,
            out_shape=e_shapes,
        )(ea_pad, p['le_w'], p['le_b'].reshape(1, din))

        d_out = p['n1_w'].shape[1]
        mlp_w = (p['n1_w'], p['n1_b'].reshape(1, d_out),
                 p['nbn_g'].reshape(1, d_out), p['nbn_b'].reshape(1, d_out),
                 p['n2_w'], p['n2_b'].reshape(1, d_out),
                 p['eps'].reshape(1, 1))
        if esplit:
            agg_a, agg_b = _sc_edge_agg_esplit(
                hn_outs[0], e_outs[0], src_re, dst_re,
                n_nodes=n, n_edges_pad=e_pad)
            z = _tc_call(
                functools.partial(_mlp_body_full, dl=din),
                jax.ShapeDtypeStruct((n, d_out), f32),
                hn_outs[0], agg_a, agg_b, *mlp_w)
        else:
            agg_l, agg_r = _sc_edge_agg_fsplit(
                hn_outs[0], hn_outs[1], e_outs[0], e_outs[1], src_rf, dst_rf,
                n_nodes=n, n_edges_pad=e_pad)
            z = _tc_call(
                functools.partial(_mlp_body_split, dl=dl),
                jax.ShapeDtypeStruct((n, d_out), f32),
                hn_outs[0], hn_outs[1], agg_l, agg_r, *mlp_w)
        parts.append(z)

    out = _tc_call(
        _final_body,
        jax.ShapeDtypeStruct((n, 1), f32),
        *parts, params['fin_w'], params['fin_b'].reshape(1, 1))
    return jnp.reshape(out, (-1,))


# speedup vs baseline: 1.0540x; 1.0540x over previous
import functools

import jax
import jax.numpy as jnp
from jax import lax
from jax.experimental import pallas as pl
from jax.experimental.pallas import tpu as pltpu
from jax.experimental.pallas import tpu_sc as plsc

_NC = 2
_NS = 16
_LANES = 16
_C = 64
_IB = 32
_ZR = 16
_LW = 128
_TRASH = 16


def _pad_cols(a, width):
    if a.shape[1] == width:
        return a
    return jnp.concatenate(
        [a, jnp.zeros((a.shape[0], width - a.shape[1]), a.dtype)], axis=1)


def _bn_body(*refs, nparts, leaky, dl):
    parts = refs[:nparts]
    g, b = refs[nparts], refs[nparts + 1]
    outs = refs[nparts + 2:]
    h = jnp.concatenate([p[...] for p in parts], axis=1)
    m = jnp.mean(h, axis=0, keepdims=True)
    v = jnp.mean((h - m) ** 2, axis=0, keepdims=True)
    hn = (h - m) * lax.rsqrt(v + 1e-5) * g[...] + b[...]
    if leaky:
        hn = jnp.where(hn >= 0, hn, 0.01 * hn)
    if len(outs) == 1:
        outs[0][...] = _pad_cols(hn, outs[0].shape[1])
    else:
        lw = outs[0].shape[1]
        outs[0][...] = _pad_cols(hn[:, :dl], lw)
        outs[1][...] = _pad_cols(hn[:, dl:], lw)


def _eproj_body(*refs, dl):
    a_ref, w_ref, b_ref = refs[:3]
    outs = refs[3:]
    e = jnp.dot(a_ref[...], w_ref[...], preferred_element_type=jnp.float32)
    e = e + b_ref[...]
    if len(outs) == 1:
        outs[0][...] = _pad_cols(e, outs[0].shape[1])
    else:
        lw = outs[0].shape[1]
        outs[0][...] = _pad_cols(e[:, :dl], lw)
        outs[1][...] = _pad_cols(e[:, dl:], lw)


def _mlp_core(hn, ag, eps, w1, b1, g, b, w2, b2, out):
    z = (1.0 + eps) * hn + ag
    z = jnp.dot(z, w1[...], preferred_element_type=jnp.float32) + b1[...]
    m = jnp.mean(z, axis=0, keepdims=True)
    v = jnp.mean((z - m) ** 2, axis=0, keepdims=True)
    z = (z - m) * lax.rsqrt(v + 1e-5) * g[...] + b[...]
    z = jnp.where(z >= 0, z, 0.01 * z)
    out[...] = jnp.dot(z, w2[...], preferred_element_type=jnp.float32) + b2[...]


def _mlp_body_split(hnl, hnr, agl, agr, w1, b1, g, b, w2, b2, eps_ref, out,
                    *, dl):
    hn = jnp.concatenate([hnl[:, :dl], hnr[:, :dl]], axis=1)
    ag = jnp.concatenate([agl[:, :dl], agr[:, :dl]], axis=1)
    _mlp_core(hn, ag, eps_ref[0, 0], w1, b1, g, b, w2, b2, out)


def _mlp_body_full(hn_ref, aga, agb, w1, b1, g, b, w2, b2, eps_ref, out,
                   *, dl):
    hn = hn_ref[:, :dl]
    ag = aga[:, :dl] + agb[:, :dl]
    _mlp_core(hn, ag, eps_ref[0, 0], w1, b1, g, b, w2, b2, out)


def _final_body(*refs):
    parts, w, b, out = refs[:-3], refs[-3], refs[-2], refs[-1]
    h = jnp.concatenate([p[...] for p in parts], axis=1)
    out[...] = jnp.dot(h, w[...], preferred_element_type=jnp.float32) + b[...]


def _zero_accum(s, zb_v, agg_sh, rows_per_tile, rows_rem, row0):
    zero = jnp.zeros((_LANES,), jnp.float32)

    def zrow(r, _):
        for j in range(_LW // _LANES):
            zb_v[r, pl.ds(j * _LANES, _LANES)] = zero
        return 0

    lax.fori_loop(0, _ZR, zrow, 0)

    def zcp(k, _):
        pltpu.sync_copy(zb_v, agg_sh.at[pl.ds(row0 + k * _ZR, _ZR)])
        return 0

    lax.fori_loop(0, rows_per_tile // _ZR, zcp, 0)

    if rows_rem:
        @pl.when(s == 0)
        def _():
            def zcp_rem(k, _):
                pltpu.sync_copy(
                    zb_v, agg_sh.at[pl.ds(rows_per_tile * _NS + k * _ZR, _ZR)])
                return 0
            lax.fori_loop(0, rows_rem // _ZR, zcp_rem, 0)


def _run_edges(hn_h, e_h, agg_sh, idx_src, idx_dst, src_v, dst_v,
               msgs, esems, gsems, ssems, ebase, nblocks):
    def block(bb, _):
        pltpu.sync_copy(idx_src(bb), src_v)
        pltpu.sync_copy(idx_dst(bb), dst_v)
        base = ebase + bb * _IB * _C

        def eload(j, b):
            pltpu.async_copy(e_h.at[pl.ds(base + j * _C, _C)],
                             msgs[b], esems[b])

        def ewait(j, b):
            pltpu.make_async_copy(e_h.at[pl.ds(base + j * _C, _C)],
                                  msgs[b], esems[b]).wait()

        def gissue(j, b):
            pltpu.async_copy(hn_h.at[src_v.at[j]], msgs[b], gsems[b],
                             add=True)

        def gwait(j, b):
            pltpu.make_async_copy(hn_h.at[src_v.at[j]], msgs[b],
                                  gsems[b]).wait()

        def sissue(j, b):
            pltpu.async_copy(msgs[b], agg_sh.at[dst_v.at[j]], ssems[b],
                             add=True)

        def swait(j, b):
            pltpu.make_async_copy(msgs[b], agg_sh.at[dst_v.at[j]],
                                  ssems[b]).wait()

        def relu(b):
            m = msgs[b]

            @plsc.parallel_loop(0, _C, unroll=4)
            def _(r):
                for q in range(_LW // _LANES):
                    sl = pl.ds(q * _LANES, _LANES)
                    m[r, sl] = jnp.maximum(m[r, sl], 0.0)

        eload(0, 0)
        eload(1, 1)
        eload(2, 2)
        ewait(0, 0)
        gissue(0, 0)
        ewait(1, 1)
        gissue(1, 1)

        for j in range(4):
            u = j % 4
            gwait(j, u)
            if j >= 1:
                swait(j - 1, (u + 3) % 4)
            if j + 3 < _IB:
                eload(j + 3, (u + 3) % 4)
            if j + 2 < _IB:
                ewait(j + 2, (u + 2) % 4)
                gissue(j + 2, (u + 2) % 4)
            relu(u)
            sissue(j, u)

        def quad(t, _):
            for u in range(4):
                j = 4 * t + u
                gwait(j, u)
                swait(j - 1, (u + 3) % 4)
                eload(j + 3, (u + 3) % 4)
                ewait(j + 2, (u + 2) % 4)
                gissue(j + 2, (u + 2) % 4)
                relu(u)
                sissue(j, u)
            return 0

        lax.fori_loop(1, _IB // 4 - 1, quad, 0)

        for j in range(_IB - 4, _IB):
            u = j % 4
            gwait(j, u)
            if j + 3 < _IB:
                swait(j - 1, (u + 3) % 4)
                eload(j + 3, (u + 3) % 4)
            if j + 2 < _IB:
                ewait(j + 2, (u + 2) % 4)
                gissue(j + 2, (u + 2) % 4)
            relu(u)
            sissue(j, u)

        for j in range(_IB - 4, _IB):
            swait(j, j % 4)
        return 0

    lax.fori_loop(0, nblocks, block, 0)


def _sc_body_fsplit(hn_l, hn_r, e_l, e_r, src_r, dst_r, agg_l, agg_r,
                    src_v, dst_v, msg0, msg1, msg2, msg3, zb_v, agg_sh,
                    es0, es1, es2, es3, gs0, gs1, gs2, gs3,
                    ss0, ss1, ss2, ss3, *, n_nodes, steps):
    c = lax.axis_index("c")
    s = lax.axis_index("s")
    rows_per_tile = (n_nodes // (_NS * 8)) * 8
    rows_rem = n_nodes - rows_per_tile * _NS
    row0 = s * rows_per_tile
    ebase = s * (steps * _C)
    nblocks = steps // _IB
    msgs = (msg0, msg1, msg2, msg3)
    esems = (es0, es1, es2, es3)
    gsems = (gs0, gs1, gs2, gs3)
    ssems = (ss0, ss1, ss2, ss3)

    _zero_accum(s, zb_v, agg_sh, rows_per_tile, rows_rem, row0)
    plsc.subcore_barrier()

    def run(hn_h, e_h, agg_h):
        _run_edges(hn_h, e_h, agg_sh,
                   lambda bb: src_r.at[s, bb], lambda bb: dst_r.at[s, bb],
                   src_v, dst_v, msgs, esems, gsems, ssems, ebase, nblocks)
        plsc.subcore_barrier()
        pltpu.sync_copy(agg_sh.at[pl.ds(row0, rows_per_tile)],
                        agg_h.at[pl.ds(row0, rows_per_tile)])
        if rows_rem:
            @pl.when(s == 0)
            def _():
                base = rows_per_tile * _NS
                pltpu.sync_copy(agg_sh.at[pl.ds(base, rows_rem)],
                                agg_h.at[pl.ds(base, rows_rem)])

    @pl.when(c == 0)
    def _():
        run(hn_l, e_l, agg_l)

    @pl.when(c == 1)
    def _():
        run(hn_r, e_r, agg_r)


def _sc_body_esplit(hn, e, src_r, dst_r, agg_a, agg_b,
                    src_v, dst_v, msg0, msg1, msg2, msg3, zb_v, agg_sh,
                    es0, es1, es2, es3, gs0, gs1, gs2, gs3,
                    ss0, ss1, ss2, ss3, *, n_nodes, steps):
    c = lax.axis_index("c")
    s = lax.axis_index("s")
    rows_per_tile = (n_nodes // (_NS * 8)) * 8
    rows_rem = n_nodes - rows_per_tile * _NS
    row0 = s * rows_per_tile
    ebase = c * (_NS * steps * _C) + s * (steps * _C)
    nblocks = steps // _IB
    msgs = (msg0, msg1, msg2, msg3)
    esems = (es0, es1, es2, es3)
    gsems = (gs0, gs1, gs2, gs3)
    ssems = (ss0, ss1, ss2, ss3)

    _zero_accum(s, zb_v, agg_sh, rows_per_tile, rows_rem, row0)
    plsc.subcore_barrier()

    def run(agg_h):
        _run_edges(hn, e, agg_sh,
                   lambda bb: src_r.at[c, s, bb],
                   lambda bb: dst_r.at[c, s, bb],
                   src_v, dst_v, msgs, esems, gsems, ssems, ebase, nblocks)
        plsc.subcore_barrier()
        pltpu.sync_copy(agg_sh.at[pl.ds(row0, rows_per_tile)],
                        agg_h.at[pl.ds(row0, rows_per_tile)])
        if rows_rem:
            @pl.when(s == 0)
            def _():
                base = rows_per_tile * _NS
                pltpu.sync_copy(agg_sh.at[pl.ds(base, rows_rem)],
                                agg_h.at[pl.ds(base, rows_rem)])

    @pl.when(c == 0)
    def _():
        run(agg_a)

    @pl.when(c == 1)
    def _():
        run(agg_b)


def _sc_scratch(n_nodes):
    f32 = jnp.float32
    return [
        pltpu.VMEM((_IB, _C), jnp.int32),
        pltpu.VMEM((_IB, _C), jnp.int32),
        pltpu.VMEM((_C, _LW), f32),
        pltpu.VMEM((_C, _LW), f32),
        pltpu.VMEM((_C, _LW), f32),
        pltpu.VMEM((_C, _LW), f32),
        pltpu.VMEM((_ZR, _LW), f32),
        pltpu.VMEM_SHARED((n_nodes + _TRASH, _LW), f32),
    ] + [pltpu.SemaphoreType.DMA] * 12


def _sc_edge_agg_fsplit(hn_l, hn_r, e_l, e_r, src_r, dst_r, *, n_nodes,
                        n_edges_pad):
    steps = n_edges_pad // (_NS * _C)
    mesh = plsc.VectorSubcoreMesh(core_axis_name="c", subcore_axis_name="s")
    f32 = jnp.float32
    return pl.kernel(
        functools.partial(_sc_body_fsplit, n_nodes=n_nodes, steps=steps),
        out_type=(jax.ShapeDtypeStruct((n_nodes, _LW), f32),
                  jax.ShapeDtypeStruct((n_nodes, _LW), f32)),
        mesh=mesh,
        scratch_types=_sc_scratch(n_nodes),
    )(hn_l, hn_r, e_l, e_r, src_r, dst_r)


def _sc_edge_agg_esplit(hn, e, src_r, dst_r, *, n_nodes, n_edges_pad):
    steps = n_edges_pad // (_NC * _NS * _C)
    mesh = plsc.VectorSubcoreMesh(core_axis_name="c", subcore_axis_name="s")
    f32 = jnp.float32
    return pl.kernel(
        functools.partial(_sc_body_esplit, n_nodes=n_nodes, steps=steps),
        out_type=(jax.ShapeDtypeStruct((n_nodes, _LW), f32),
                  jax.ShapeDtypeStruct((n_nodes, _LW), f32)),
        mesh=mesh,
        scratch_types=_sc_scratch(n_nodes),
    )(hn, e, src_r, dst_r)


def _tc_call(body, out_shapes, *args):
    return pl.pallas_call(body, out_shape=out_shapes)(*args)


def kernel(x, edge_index, edge_attr, params):
    n, d_feat = x.shape
    e_cnt = edge_index.shape[1]
    f32 = jnp.float32

    chunk = _NC * _NS * _IB * _C
    e_pad = ((e_cnt + chunk - 1) // chunk) * chunk
    steps_f = e_pad // (_NS * _C)
    steps_e = e_pad // (_NC * _NS * _C)
    src_flat = jnp.concatenate(
        [edge_index[0].astype(jnp.int32),
         jnp.zeros((e_pad - e_cnt,), jnp.int32)])
    dst_flat = jnp.concatenate(
        [edge_index[1].astype(jnp.int32),
         jnp.full((e_pad - e_cnt,), n, jnp.int32)])
    src_rf = src_flat.reshape(_NS, steps_f // _IB, _IB, _C)
    dst_rf = dst_flat.reshape(_NS, steps_f // _IB, _IB, _C)
    src_re = src_flat.reshape(_NC, _NS, steps_e // _IB, _IB, _C)
    dst_re = dst_flat.reshape(_NC, _NS, steps_e // _IB, _IB, _C)
    ea_pad = jnp.concatenate(
        [edge_attr, jnp.zeros((e_pad - e_cnt, edge_attr.shape[1]), f32)])

    e_outs_all = []
    for i, p in enumerate(params['layers']):
        din = p['le_w'].shape[1]
        dl = din // 2
        n_hn = 1 if din == _LW else 2
        be = 4096
        e_shapes = tuple(jax.ShapeDtypeStruct((e_pad, _LW), f32)
                         for _ in range(n_hn))
        e_outs_all.append(pl.pallas_call(
            functools.partial(_eproj_body, dl=dl),
            grid=(e_pad // be,),
            in_specs=[
                pl.BlockSpec((be, edge_attr.shape[1]), lambda j: (j, 0)),
                pl.BlockSpec((edge_attr.shape[1], din), lambda j: (0, 0)),
                pl.BlockSpec((1, din), lambda j: (0, 0)),
            ],
            out_specs=[pl.BlockSpec((be, _LW), lambda j: (j, 0))] * n_hn,
            out_shape=e_shapes,
        )(ea_pad, p['le_w'], p['le_b'].reshape(1, din)))

    parts = [x]
    for i, p in enumerate(params['layers']):
        din = sum(q.shape[1] for q in parts)
        dl = din // 2
        esplit = (din == _LW)
        n_hn = 1 if esplit else 2
        g2 = p['bn_g'].reshape(1, din)
        b2 = p['bn_b'].reshape(1, din)
        hn_shapes = tuple(jax.ShapeDtypeStruct((n, _LW), f32)
                          for _ in range(n_hn))
        hn_outs = _tc_call(
            functools.partial(_bn_body, nparts=len(parts), leaky=(i > 0),
                              dl=dl),
            hn_shapes, *parts, g2, b2)
        e_outs = e_outs_all[i]

        d_out = p['n1_w'].shape[1]
        mlp_w = (p['n1_w'], p['n1_b'].reshape(1, d_out),
                 p['nbn_g'].reshape(1, d_out), p['nbn_b'].reshape(1, d_out),
                 p['n2_w'], p['n2_b'].reshape(1, d_out),
                 p['eps'].reshape(1, 1))
        if esplit:
            agg_a, agg_b = _sc_edge_agg_esplit(
                hn_outs[0], e_outs[0], src_re, dst_re,
                n_nodes=n, n_edges_pad=e_pad)
            z = _tc_call(
                functools.partial(_mlp_body_full, dl=din),
                jax.ShapeDtypeStruct((n, d_out), f32),
                hn_outs[0], agg_a, agg_b, *mlp_w)
        else:
            agg_l, agg_r = _sc_edge_agg_fsplit(
                hn_outs[0], hn_outs[1], e_outs[0], e_outs[1], src_rf, dst_rf,
                n_nodes=n, n_edges_pad=e_pad)
            z = _tc_call(
                functools.partial(_mlp_body_split, dl=dl),
                jax.ShapeDtypeStruct((n, d_out), f32),
                hn_outs[0], hn_outs[1], agg_l, agg_r, *mlp_w)
        parts.append(z)

    out = _tc_call(
        _final_body,
        jax.ShapeDtypeStruct((n, 1), f32),
        *parts, params['fin_w'], params['fin_b'].reshape(1, 1))
    return jnp.reshape(out, (-1,))
